# Initial kernel scaffold; baseline (speedup 1.0000x reference)
import jax
import jax.numpy as jnp
from jax.experimental import pallas as pl


def kernel(src, pos, neg, batch_size, u_weight, v_weight):
    raise NotImplementedError("write your pallas kernel here")


# speedup vs baseline: 5.1956x; 5.1956x over previous
import functools

import jax
import jax.numpy as jnp
from jax import lax
from jax.experimental import pallas as pl
from jax.experimental.pallas import tpu as pltpu
from jax.experimental.pallas import tpu_sc as plsc

B = 16384
D = 64
K = 20
CH = K + 1


def _sc_info():
    try:
        info = plsc.get_sparse_core_info()
        return info.num_cores, info.num_subcores
    except Exception:
        return 2, 16


def _dot_partial(src_v, buf, b):
    acc = src_v[b, pl.ds(0, 16)] * buf[b, pl.ds(0, 16)]
    for j in range(1, 4):
        acc = acc + src_v[b, pl.ds(16 * j, 16)] * buf[b, pl.ds(16 * j, 16)]
    return acc


def _make_sc_kernel(nc, ns):
    nw = nc * ns
    bpt = B // nw
    mesh = plsc.VectorSubcoreMesh(core_axis_name="c", subcore_axis_name="s")

    def body(u_hbm, v_hbm, sidx_hbm, vidx_hbm, psump_hbm, psumn_hbm,
             idx_s, idx_v, src_v, buf0, buf1, psum_p, psum_n,
             sem_s, sem0, sem1, sem_out):
        wid = lax.axis_index("s") * nc + lax.axis_index("c")
        base = wid * bpt
        pltpu.sync_copy(sidx_hbm.at[pl.ds(base, bpt)], idx_s)
        pltpu.sync_copy(vidx_hbm.at[pl.ds(wid * (CH * bpt), CH * bpt)], idx_v)
        cp_s = pltpu.async_copy(u_hbm.at[idx_s], src_v, sem_s)
        bufs = (buf0, buf1)
        sems = (sem0, sem1)
        cp = pltpu.async_copy(v_hbm.at[idx_v.at[pl.ds(0, bpt)]], buf0, sem0)
        cp_s.wait()

        for c in range(CH):
            if c + 1 < CH:
                cp_next = pltpu.async_copy(
                    v_hbm.at[idx_v.at[pl.ds((c + 1) * bpt, bpt)]],
                    bufs[(c + 1) % 2], sems[(c + 1) % 2])
            cp.wait()
            buf = bufs[c % 2]
            psum = psum_p if c == 0 else psum_n
            overwrite = c <= 1

            def chunk_body(b, carry, buf=buf, psum=psum, overwrite=overwrite):
                contrib = _dot_partial(src_v, buf, b)
                if overwrite:
                    psum[pl.ds(b * 16, 16)] = contrib
                else:
                    plsc.addupdate(psum.at[pl.ds(b * 16, 16)], contrib)
                return carry

            lax.fori_loop(0, bpt, chunk_body, 0)
            if c == 0:
                cp_out = pltpu.async_copy(
                    psum_p, psump_hbm.at[pl.ds(base * 16, bpt * 16)], sem_out)
            if c + 1 < CH:
                cp = cp_next
        cp_out.wait()
        pltpu.sync_copy(psum_n, psumn_hbm.at[pl.ds(base * 16, bpt * 16)])

    kern = pl.kernel(
        body,
        compiler_params=pltpu.CompilerParams(use_tc_tiling_on_sc=False),
        out_type=[
            jax.ShapeDtypeStruct((B * 16,), jnp.float32),
            jax.ShapeDtypeStruct((B * 16,), jnp.float32),
        ],
        mesh=mesh,
        scratch_types=[
            pltpu.VMEM((bpt,), jnp.int32),
            pltpu.VMEM((CH * bpt,), jnp.int32),
            pltpu.VMEM((bpt, D), jnp.float32),
            pltpu.VMEM((bpt, D), jnp.float32),
            pltpu.VMEM((bpt, D), jnp.float32),
            pltpu.VMEM((bpt * 16,), jnp.float32),
            pltpu.VMEM((bpt * 16,), jnp.float32),
            pltpu.SemaphoreType.DMA,
            pltpu.SemaphoreType.DMA,
            pltpu.SemaphoreType.DMA,
            pltpu.SemaphoreType.DMA,
        ],
    )
    return kern


def _loss_body(p_ref, n_ref, o_ref):
    lane = lax.broadcasted_iota(jnp.int32, (128, 8), 0)
    grp = lax.broadcasted_iota(jnp.int32, (128, 8), 1)
    m = (lane // 16 == grp).astype(jnp.float32)
    s = jnp.dot(p_ref[...], m, preferred_element_type=jnp.float32)
    n = jnp.dot(n_ref[...], m, preferred_element_type=jnp.float32)
    ls = jnp.minimum(s, 0.0) - jnp.log(1.0 + jnp.exp(-jnp.abs(s)))
    ln = jnp.minimum(-n, 0.0) - jnp.log(1.0 + jnp.exp(-jnp.abs(n)))
    total = jnp.sum(ls) + jnp.sum(ln)
    o_ref[...] = jnp.broadcast_to(total, (1, 1))


def kernel(src, pos, neg, batch_size, u_weight, v_weight):
    nc, ns = _sc_info()
    src_i = src.astype(jnp.int32)
    nw = nc * ns
    bpt = B // nw
    vidx = jnp.concatenate(
        [pos.astype(jnp.int32)[None, :], neg.astype(jnp.int32).T], axis=0)
    vidx = vidx.reshape(CH, nw, bpt).transpose(1, 0, 2).reshape(-1)
    psum_p, psum_n = _make_sc_kernel(nc, ns)(u_weight, v_weight, src_i, vidx)

    total = pl.pallas_call(
        _loss_body,
        out_shape=jax.ShapeDtypeStruct((1, 1), jnp.float32),
    )(psum_p.reshape(B // 8, 128), psum_n.reshape(B // 8, 128))
    return -total[0, 0] / batch_size
